# baseline (speedup 1.0000x reference)
import functools

import jax
import jax.numpy as jnp
from jax import lax
from jax.experimental import pallas as pl
from jax.experimental.pallas import tpu as pltpu
from jax.experimental.pallas import tpu_sc as plsc

B = 16384
EMB = 128
EPS = 1e-5

NUM_WORKERS = 32
ROWS_PER_W = B // NUM_WORKERS
CHUNK = 128
NCHUNK = ROWS_PER_W // CHUNK


def _sc_gather_body(uidx_hbm, iidx_hbm, utab_hbm, itab_hbm,
                    x_out, uidx_v, iidx_v, rows0, rows1, gsem, wsem0, wsem1):
    wid = lax.axis_index("s") * 2 + lax.axis_index("c")
    base = wid * ROWS_PER_W
    pltpu.sync_copy(uidx_hbm.at[pl.ds(base, ROWS_PER_W)], uidx_v)
    pltpu.sync_copy(iidx_hbm.at[pl.ds(base, ROWS_PER_W)], iidx_v)
    bufs = (rows0, rows1)
    wsems = (wsem0, wsem1)
    writes = [None, None]
    for s in range(2 * NCHUNK):
        if s < NCHUNK:
            idx_v, tab_hbm, col, j = uidx_v, utab_hbm, 0, s
        else:
            idx_v, tab_hbm, col, j = iidx_v, itab_hbm, EMB, s - NCHUNK
        buf, wsem = bufs[s % 2], wsems[s % 2]
        if writes[s % 2] is not None:
            writes[s % 2].wait()
        pltpu.async_copy(tab_hbm.at[idx_v.at[pl.ds(j * CHUNK, CHUNK)]],
                         buf, gsem).wait()
        writes[s % 2] = pltpu.async_copy(
            buf, x_out.at[pl.ds(base + j * CHUNK, CHUNK), pl.ds(col, EMB)],
            wsem)
    for w in writes:
        w.wait()


@functools.cache
def _make_gather():
    return pl.kernel(
        _sc_gather_body,
        mesh=plsc.VectorSubcoreMesh(core_axis_name="c", subcore_axis_name="s"),
        out_type=jax.ShapeDtypeStruct((B, 2 * EMB), jnp.float32),
        scratch_types=[pltpu.VMEM((ROWS_PER_W,), jnp.int32),
                       pltpu.VMEM((ROWS_PER_W,), jnp.int32),
                       pltpu.VMEM((CHUNK, EMB), jnp.float32),
                       pltpu.VMEM((CHUNK, EMB), jnp.float32),
                       pltpu.SemaphoreType.DMA,
                       pltpu.SemaphoreType.DMA,
                       pltpu.SemaphoreType.DMA],
    )


def _bn_fold(x, g, beta):
    mu = jnp.mean(x, axis=0, keepdims=True)
    var = jnp.mean(x * x, axis=0, keepdims=True) - mu * mu
    a = g * lax.rsqrt(var + EPS)
    c = beta - a * mu
    return jnp.maximum(a * x + c, 0.0)


def _mlp_body(x2, W1, b1, g1, beta1, W23, b23, g3, beta3, Wo, bo, out):
    x = jnp.dot(x2[...], W1[...]) + b1[...]
    y = _bn_fold(x, g1[...], beta1[...])
    h = jnp.dot(y, W23[...]) + b23[...]
    hh = _bn_fold(h, g3[...], beta3[...])
    out[...] = (jnp.dot(hh, Wo[...]) + bo[...]).reshape(B)


_mlp = pl.pallas_call(
    _mlp_body,
    out_shape=jax.ShapeDtypeStruct((B,), jnp.float32),
)


def _mlp_out_2d(*args):
    return _mlp(*args).reshape(B, 1)


def kernel(user_input, item_input, user_table, item_table,
           uW1, ub1, ug1, ubeta1, uW2, ub2,
           iW1, ib1, ig1, ibeta1, iW2, ib2,
           W3, b3, g3, beta3, Wo, bo):
    uidx = user_input.astype(jnp.int32)
    iidx = item_input.astype(jnp.int32)
    x2 = _make_gather()(uidx, iidx, user_table, item_table)

    W1 = jnp.concatenate(
        [jnp.concatenate([uW1, jnp.zeros_like(uW1)], axis=1),
         jnp.concatenate([jnp.zeros_like(iW1), iW1], axis=1)], axis=0)
    W2 = jnp.concatenate(
        [jnp.concatenate([uW2, jnp.zeros_like(uW2)], axis=1),
         jnp.concatenate([jnp.zeros_like(iW2), iW2], axis=1)], axis=0)
    W23 = W2 @ W3
    b23 = jnp.concatenate([ub2, ib2]) @ W3 + b3
    r = lambda v: v.reshape(1, -1)
    return _mlp_out_2d(x2, W1, r(jnp.concatenate([ub1, ib1])),
                r(jnp.concatenate([ug1, ig1])),
                r(jnp.concatenate([ubeta1, ibeta1])),
                W23, r(b23), r(g3), r(beta3), Wo, r(bo))


# speedup vs baseline: 1.0534x; 1.0534x over previous
import functools

import jax
import jax.numpy as jnp
from jax import lax
from jax.experimental import pallas as pl
from jax.experimental.pallas import tpu as pltpu
from jax.experimental.pallas import tpu_sc as plsc

B = 16384
EMB = 128
EPS = 1e-5

NUM_WORKERS = 32
ROWS_PER_W = B // NUM_WORKERS
CHUNK = 128
NCHUNK = ROWS_PER_W // CHUNK


HALF = ROWS_PER_W // 2


def _sc_gather_body(uidx_hbm, iidx_hbm, utab_hbm, itab_hbm,
                    x_out, uidx_v, iidx_v, rows_a, rows_b, gsem, wsem):
    wid = lax.axis_index("s") * 2 + lax.axis_index("c")
    base = wid * ROWS_PER_W
    pltpu.sync_copy(uidx_hbm.at[pl.ds(base, ROWS_PER_W)], uidx_v)
    pltpu.sync_copy(iidx_hbm.at[pl.ds(base, ROWS_PER_W)], iidx_v)
    gathers = [pltpu.async_copy(
        utab_hbm.at[uidx_v.at[pl.ds(j * CHUNK, CHUNK)]],
        rows_a.at[pl.ds(j * CHUNK, CHUNK)], gsem) for j in range(NCHUNK)]
    for g in gathers:
        g.wait()
    w_a = pltpu.async_copy(
        rows_a, x_out.at[pl.ds(base, ROWS_PER_W), pl.ds(0, EMB)], wsem)
    gathers = [pltpu.async_copy(
        itab_hbm.at[iidx_v.at[pl.ds(j * CHUNK, CHUNK)]],
        rows_b.at[pl.ds(j * CHUNK, CHUNK)], gsem) for j in range(2)]
    for g in gathers:
        g.wait()
    w_a.wait()
    w_b = pltpu.async_copy(
        rows_b, x_out.at[pl.ds(base, HALF), pl.ds(EMB, EMB)], wsem)
    gathers = [pltpu.async_copy(
        itab_hbm.at[iidx_v.at[pl.ds((2 + j) * CHUNK, CHUNK)]],
        rows_a.at[pl.ds(j * CHUNK, CHUNK)], gsem) for j in range(2)]
    for g in gathers:
        g.wait()
    w_b.wait()
    pltpu.sync_copy(rows_a.at[pl.ds(0, HALF)],
                    x_out.at[pl.ds(base + HALF, HALF), pl.ds(EMB, EMB)])


@functools.cache
def _make_gather():
    return pl.kernel(
        _sc_gather_body,
        mesh=plsc.VectorSubcoreMesh(core_axis_name="c", subcore_axis_name="s"),
        out_type=jax.ShapeDtypeStruct((B, 2 * EMB), jnp.float32),
        scratch_types=[pltpu.VMEM((ROWS_PER_W,), jnp.int32),
                       pltpu.VMEM((ROWS_PER_W,), jnp.int32),
                       pltpu.VMEM((ROWS_PER_W, EMB), jnp.float32),
                       pltpu.VMEM((HALF, EMB), jnp.float32),
                       pltpu.SemaphoreType.DMA,
                       pltpu.SemaphoreType.DMA],
    )


def _bn_fold(x, g, beta):
    mu = jnp.mean(x, axis=0, keepdims=True)
    var = jnp.mean(x * x, axis=0, keepdims=True) - mu * mu
    a = g * lax.rsqrt(var + EPS)
    c = beta - a * mu
    return jnp.maximum(a * x + c, 0.0)


def _mlp_body(x2, W1, b1, g1, beta1, W23, b23, g3, beta3, Wo, bo, out):
    x = jnp.dot(x2[...], W1[...]) + b1[...]
    y = _bn_fold(x, g1[...], beta1[...])
    h = jnp.dot(y, W23[...]) + b23[...]
    hh = _bn_fold(h, g3[...], beta3[...])
    out[...] = (jnp.dot(hh, Wo[...]) + bo[...]).reshape(B)


_mlp = pl.pallas_call(
    _mlp_body,
    out_shape=jax.ShapeDtypeStruct((B,), jnp.float32),
)


def _mlp_out_2d(*args):
    return _mlp(*args).reshape(B, 1)


def kernel(user_input, item_input, user_table, item_table,
           uW1, ub1, ug1, ubeta1, uW2, ub2,
           iW1, ib1, ig1, ibeta1, iW2, ib2,
           W3, b3, g3, beta3, Wo, bo):
    uidx = user_input.astype(jnp.int32)
    iidx = item_input.astype(jnp.int32)
    x2 = _make_gather()(uidx, iidx, user_table, item_table)

    W1 = jnp.concatenate(
        [jnp.concatenate([uW1, jnp.zeros_like(uW1)], axis=1),
         jnp.concatenate([jnp.zeros_like(iW1), iW1], axis=1)], axis=0)
    W2 = jnp.concatenate(
        [jnp.concatenate([uW2, jnp.zeros_like(uW2)], axis=1),
         jnp.concatenate([jnp.zeros_like(iW2), iW2], axis=1)], axis=0)
    W23 = W2 @ W3
    b23 = jnp.concatenate([ub2, ib2]) @ W3 + b3
    r = lambda v: v.reshape(1, -1)
    return _mlp_out_2d(x2, W1, r(jnp.concatenate([ub1, ib1])),
                r(jnp.concatenate([ug1, ig1])),
                r(jnp.concatenate([ubeta1, ibeta1])),
                W23, r(b23), r(g3), r(beta3), Wo, r(bo))
